# baseline (device time: 33223 ns/iter reference)
import jax
import jax.numpy as jnp
from jax import lax
from jax.experimental import pallas as pl
from jax.experimental.pallas import tpu as pltpu

N_DEV = 32


def kernel(x, w_mat):
    k_dim, k_per = x.shape
    n = w_mat.shape[1]
    m_per = k_dim // N_DEV
    rows_per_tile = m_per * k_per // 128

    def body(x_ref, w_hbm, out_ref, xpack_ref, gpack_ref, wbuf_ref,
             send_sems, recv_sems, w_sems, bar_sems):
        my_i = lax.axis_index("i")

        for k in range(5):
            dist = 1 << k
            fwd = lax.rem(my_i + dist, N_DEV)
            sem = pltpu.get_barrier_semaphore() if k == 0 else bar_sems.at[k - 1]
            pl.semaphore_signal(
                sem, inc=1,
                device_id=(fwd,), device_id_type=pl.DeviceIdType.MESH,
            )
            pl.semaphore_wait(sem, 1)

        for j in range(N_DEV):
            pltpu.make_async_copy(
                w_hbm.at[pl.ds(j * m_per, m_per), :],
                wbuf_ref.at[j],
                w_sems.at[j],
            ).start()

        for j in range(N_DEV):
            xpack_ref[j] = jnp.concatenate(
                [x_ref[pl.ds(j * m_per, m_per // 2), :],
                 x_ref[pl.ds(j * m_per + m_per // 2, m_per // 2), :]],
                axis=1,
            )

        gpack_ref[my_i] = xpack_ref[my_i]

        sends = []
        for d in range(1, N_DEV):
            j = lax.rem(my_i + d, N_DEV)
            rdma = pltpu.make_async_remote_copy(
                src_ref=xpack_ref.at[j],
                dst_ref=gpack_ref.at[my_i],
                send_sem=send_sems.at[d],
                recv_sem=recv_sems.at[my_i],
                device_id=(j,),
                device_id_type=pl.DeviceIdType.MESH,
            )
            rdma.start()
            sends.append(rdma)

        for d in range(N_DEV):
            j = lax.rem(my_i + d, N_DEV)

            pltpu.make_async_copy(
                w_hbm.at[pl.ds(j * m_per, m_per), :],
                wbuf_ref.at[j],
                w_sems.at[j],
            ).wait()

            if d > 0:
                recv = pltpu.make_async_remote_copy(
                    src_ref=gpack_ref.at[j],
                    dst_ref=gpack_ref.at[j],
                    send_sem=send_sems.at[0],
                    recv_sem=recv_sems.at[j],
                    device_id=(my_i,),
                    device_id_type=pl.DeviceIdType.MESH,
                )
                recv.wait_recv()

            packed = gpack_ref[j]
            tile = jnp.concatenate(
                [packed[:, :k_per], packed[:, k_per:]], axis=0)
            part = jnp.dot(
                tile, wbuf_ref[j],
                preferred_element_type=jnp.float32,
            )
            if d == 0:
                out_ref[:, :] = part
            else:
                out_ref[:, :] += part

        out_ref[:, :] = jnp.maximum(out_ref[:, :], 0.0)

        for rdma in sends:
            rdma.wait_send()

    return pl.pallas_call(
        body,
        out_shape=jax.ShapeDtypeStruct((m_per, n), jnp.float32),
        in_specs=[
            pl.BlockSpec(memory_space=pltpu.VMEM),
            pl.BlockSpec(memory_space=pl.ANY),
        ],
        out_specs=pl.BlockSpec(memory_space=pltpu.VMEM),
        scratch_shapes=[
            pltpu.VMEM((N_DEV, m_per * k_per // 128, 128), jnp.float32),
            pltpu.VMEM((N_DEV, m_per * k_per // 128, 128), jnp.float32),
            pltpu.VMEM((N_DEV, m_per, n), jnp.float32),
            pltpu.SemaphoreType.DMA((N_DEV,)),
            pltpu.SemaphoreType.DMA((N_DEV,)),
            pltpu.SemaphoreType.DMA((N_DEV,)),
            pltpu.SemaphoreType.REGULAR((4,)),
        ],
        compiler_params=pltpu.CompilerParams(collective_id=0),
    )(x, w_mat)


# device time: 25608 ns/iter; 1.2974x vs baseline; 1.2974x over previous
import jax
import jax.numpy as jnp
from jax import lax
from jax.experimental import pallas as pl
from jax.experimental.pallas import tpu as pltpu

N_DEV = 32


def kernel(x, w_mat):
    k_dim, k_per = x.shape
    n = w_mat.shape[1]
    m_per = k_dim // N_DEV
    half = m_per // 2

    def body(x_hbm, w_hbm, out_hbm, xv_ref, xpack_ref, gpack_ref, wbuf_ref,
             out_ref, send_sems, recv_sems, w_sems, x_sem, out_sem):
        my_i = lax.axis_index("i")

        barrier_sem = pltpu.get_barrier_semaphore()
        pl.semaphore_signal(barrier_sem, 1)
        pl.semaphore_wait(barrier_sem, 1)

        for j in range(N_DEV):
            pltpu.make_async_copy(
                w_hbm.at[pl.ds(j * m_per, m_per), :],
                wbuf_ref.at[j],
                w_sems.at[j],
            ).start()

        xcopy = pltpu.make_async_copy(x_hbm, xv_ref, x_sem)
        xcopy.start()
        xcopy.wait()
        for j in range(N_DEV):
            xpack_ref[j] = jnp.concatenate(
                [xv_ref[pl.ds(j * m_per, half), :],
                 xv_ref[pl.ds(j * m_per + half, half), :]],
                axis=1,
            )

        gpack_ref[my_i] = xpack_ref[my_i]

        sends = []
        for d in range(1, N_DEV):
            j = lax.rem(my_i + d, N_DEV)
            rdma = pltpu.make_async_remote_copy(
                src_ref=xpack_ref.at[j],
                dst_ref=gpack_ref.at[my_i],
                send_sem=send_sems.at[d],
                recv_sem=recv_sems.at[my_i],
                device_id=(j,),
                device_id_type=pl.DeviceIdType.MESH,
            )
            rdma.start()
            sends.append(rdma)

        for d in range(N_DEV):
            j = lax.rem(my_i + d, N_DEV)

            pltpu.make_async_copy(
                w_hbm.at[pl.ds(j * m_per, m_per), :],
                wbuf_ref.at[j],
                w_sems.at[j],
            ).wait()

            if d > 0:
                recv = pltpu.make_async_remote_copy(
                    src_ref=gpack_ref.at[j],
                    dst_ref=gpack_ref.at[j],
                    send_sem=send_sems.at[0],
                    recv_sem=recv_sems.at[j],
                    device_id=(my_i,),
                    device_id_type=pl.DeviceIdType.MESH,
                )
                recv.wait_recv()

            packed = gpack_ref[j]
            tile = jnp.concatenate(
                [packed[:, :k_per], packed[:, k_per:]], axis=0)
            part = jnp.dot(
                tile, wbuf_ref[j],
                preferred_element_type=jnp.float32,
            )
            if d == 0:
                out_ref[:, :] = part
            else:
                out_ref[:, :] += part

        out_ref[:, :] = jnp.maximum(out_ref[:, :], 0.0)

        ocopy = pltpu.make_async_copy(out_ref, out_hbm, out_sem)
        ocopy.start()
        ocopy.wait()

        for rdma in sends:
            rdma.wait_send()

    return pl.pallas_call(
        body,
        out_shape=jax.ShapeDtypeStruct((m_per, n), jnp.float32),
        in_specs=[
            pl.BlockSpec(memory_space=pl.ANY),
            pl.BlockSpec(memory_space=pl.ANY),
        ],
        out_specs=pl.BlockSpec(memory_space=pl.ANY),
        scratch_shapes=[
            pltpu.VMEM((k_dim, k_per), jnp.float32),
            pltpu.VMEM((N_DEV, half, 2 * k_per), jnp.float32),
            pltpu.VMEM((N_DEV, half, 2 * k_per), jnp.float32),
            pltpu.VMEM((N_DEV, m_per, n), jnp.float32),
            pltpu.VMEM((m_per, n), jnp.float32),
            pltpu.SemaphoreType.DMA((N_DEV,)),
            pltpu.SemaphoreType.DMA((N_DEV,)),
            pltpu.SemaphoreType.DMA((N_DEV,)),
            pltpu.SemaphoreType.DMA,
            pltpu.SemaphoreType.DMA,
        ],
        compiler_params=pltpu.CompilerParams(collective_id=0),
    )(x, w_mat)
